# baseline (speedup 1.0000x reference)
import jax
import jax.numpy as jnp
from jax import lax
from jax.experimental import pallas as pl
from jax.experimental.pallas import tpu as pltpu
from jax.experimental.pallas import tpu_sc as plsc

NUM_ITEMS = 1000000
SIDE_VOCAB = 100000
N_SIDE = 2
EMB = 64
B = 16384
NCTX = 5

NC = 2
NS = 16
L = 16
NW = NC * NS
BW = B // NW
CH = 32
NCHUNK = BW // CH
NBUF = 3
PEMB = 2 * EMB


def _widen_body(x_ref, o_ref):
    o_ref[:, 0:EMB] = jnp.swapaxes(x_ref[...], 0, 1)


def _widen_table(xT, v, bc=16384):
    return pl.pallas_call(
        _widen_body,
        grid=(pl.cdiv(v, bc),),
        in_specs=[pl.BlockSpec((EMB, bc), lambda c: (0, c))],
        out_specs=pl.BlockSpec((bc, PEMB), lambda c: (c, 0)),
        out_shape=jax.ShapeDtypeStruct((v, PEMB), jnp.float32),
    )(xT)


def _widen_side_body(x_ref, o_ref):
    o_ref[0, :, 0:EMB] = jnp.swapaxes(x_ref[0], 0, 1)


def _widen_side(sT, v, bc=16384):
    return pl.pallas_call(
        _widen_side_body,
        grid=(N_SIDE, pl.cdiv(v, bc)),
        in_specs=[pl.BlockSpec((1, EMB, bc), lambda j, c: (j, 0, c))],
        out_specs=pl.BlockSpec((1, bc, PEMB), lambda j, c: (j, c, 0)),
        out_shape=jax.ShapeDtypeStruct((N_SIDE, v, PEMB), jnp.float32),
    )(sT)


def _softmax3(w0, w1, w2):
    m = jnp.maximum(w0, jnp.maximum(w1, w2))
    e0 = jnp.exp(w0 - m)
    e1 = jnp.exp(w1 - m)
    e2 = jnp.exp(w2 - m)
    s = e0 + e1 + e2
    return e0 / s, e1 / s, e2 / s


def _body_a(ci_hbm, csi_hbm, ein_hbm, w0_hbm, w1_hbm, w2_hbm, side_hbm,
            hid_hbm, *scratch):
    per = 9
    slots = [scratch[i * per:(i + 1) * per] for i in range(NBUF)]
    wid = lax.axis_index("s") * NC + lax.axis_index("c")
    base0 = wid * BW
    iota16 = lax.iota(jnp.int32, L)

    def issue_idx(k):
        ii, is0, is1, ri, rs0, rs1, w3, hb, sem = slots[k % NBUF]
        base = base0 + k * CH
        hs = [
            pltpu.make_async_copy(ci_hbm.at[pl.ds(base, CH)], ii, sem),
            pltpu.make_async_copy(csi_hbm.at[pl.ds(base, CH)], is0, sem),
            pltpu.make_async_copy(csi_hbm.at[pl.ds(B + base, CH)], is1,
                                  sem),
        ]
        for h in hs:
            h.start()
        return hs

    def issue_gather(k, idx_pending):
        ii, is0, is1, ri, rs0, rs1, w3, hb, sem = slots[k % NBUF]
        for h in idx_pending:
            h.wait()
        hs = [
            pltpu.make_async_copy(ein_hbm.at[ii], ri, sem),
            pltpu.make_async_copy(side_hbm.at[0].at[is0], rs0, sem),
            pltpu.make_async_copy(side_hbm.at[1].at[is1], rs1, sem),
            pltpu.make_async_copy(w0_hbm.at[ii], w3.at[pl.ds(0, CH)], sem),
            pltpu.make_async_copy(w1_hbm.at[ii], w3.at[pl.ds(CH, CH)], sem),
            pltpu.make_async_copy(w2_hbm.at[ii], w3.at[pl.ds(2 * CH, CH)],
                                  sem),
        ]
        for h in hs:
            h.start()
        return hs

    def compute(k):
        ii, is0, is1, ri, rs0, rs1, w3, hb, sem = slots[k % NBUF]

        def group(g, carry):
            o = g * L
            lane = jnp.full((L,), o, jnp.int32) + iota16
            lane64 = lane * EMB
            p0, p1, p2 = _softmax3(
                w3[pl.ds(o, L)], w3[pl.ds(CH + o, L)],
                w3[pl.ds(2 * CH + o, L)])

            def dbody(d, carry2):
                dv = jnp.full((L,), d, jnp.int32)
                h = (p0 * plsc.load_gather(ri, [lane, dv])
                     + p1 * plsc.load_gather(rs0, [lane, dv])
                     + p2 * plsc.load_gather(rs1, [lane, dv]))
                plsc.store_scatter(hb, [lane64 + dv], h)
                return carry2

            lax.fori_loop(0, EMB, dbody, 0)
            return carry

        lax.fori_loop(0, CH // L, group, 0)

    pend = [issue_gather(0, issue_idx(0))]
    for k in range(1, NBUF - 1):
        pend.append(issue_gather(k, issue_idx(k)))
    for k in range(NCHUNK):
        if k + NBUF - 1 < NCHUNK:
            pend.append(issue_gather(k + NBUF - 1,
                                     issue_idx(k + NBUF - 1)))
        for h in pend.pop(0):
            h.wait()
        compute(k)
        hb = slots[k % NBUF][7]
        pltpu.sync_copy(
            hb, hid_hbm.at[pl.ds((base0 + k * CH) * EMB, CH * EMB)])


def _scratch_a():
    per_slot = [
        pltpu.VMEM((CH,), jnp.int32),
        pltpu.VMEM((CH,), jnp.int32),
        pltpu.VMEM((CH,), jnp.int32),
        pltpu.VMEM((CH, PEMB), jnp.float32),
        pltpu.VMEM((CH, PEMB), jnp.float32),
        pltpu.VMEM((CH, PEMB), jnp.float32),
        pltpu.VMEM((CH * 3,), jnp.float32),
        pltpu.VMEM((CH * EMB,), jnp.float32),
        pltpu.SemaphoreType.DMA,
    ]
    return per_slot * NBUF


CHB = 32
NBUFB = 4
NCHUNKB = BW // CHB


def _body_b(ctx_hbm, eout_hbm, hid_hbm, out_hbm, *scratch):
    per = 5
    slots = [scratch[i * per:(i + 1) * per] for i in range(NBUFB)]
    wid = lax.axis_index("s") * NC + lax.axis_index("c")
    base0 = wid * BW
    iota16 = lax.iota(jnp.int32, L)

    def issue_idx(k):
        icf, rc, hbv, ob, sem = slots[k % NBUFB]
        base = base0 + k * CHB
        hs = [pltpu.make_async_copy(
            hid_hbm.at[pl.ds(base * EMB, CHB * EMB)], hbv, sem)]
        for c in range(NCTX):
            hs.append(pltpu.make_async_copy(
                ctx_hbm.at[pl.ds(c * B + base, CHB)],
                icf.at[pl.ds(c * CHB, CHB)], sem))
        for h in hs:
            h.start()
        return hs

    def issue_gather(k, idx_pending):
        icf, rc, hbv, ob, sem = slots[k % NBUFB]
        for h in idx_pending:
            h.wait()
        hs = []
        for c in range(NCTX):
            hs.append(pltpu.make_async_copy(
                eout_hbm.at[icf.at[pl.ds(c * CHB, CHB)]],
                rc.at[pl.ds(c * CHB, CHB)], sem))
        for h in hs:
            h.start()
        return hs

    def compute(k):
        icf, rc, hbv, ob, sem = slots[k % NBUFB]
        zf = jnp.zeros((L,), jnp.float32)

        def group(g, carry):
            o = g * L
            lane = jnp.full((L,), o, jnp.int32) + iota16
            lane64 = lane * EMB

            def dbody(d, accs):
                dv = jnp.full((L,), d, jnp.int32)
                hv = plsc.load_gather(hbv, [lane64 + dv])
                return tuple(
                    accs[c] + hv * plsc.load_gather(
                        rc, [jnp.full((L,), c * CHB, jnp.int32) + lane, dv])
                    for c in range(NCTX))

            accs = lax.fori_loop(0, EMB, dbody, (zf,) * NCTX)
            for c in range(NCTX):
                sig = 1.0 / (1.0 + jnp.exp(-accs[c]))
                plsc.store_scatter(
                    ob, [jnp.full((L,), c * CHB, jnp.int32) + lane], sig)
            return carry

        lax.fori_loop(0, CHB // L, group, 0)

    pend = [issue_gather(0, issue_idx(0))]
    for k in range(1, NBUFB - 1):
        pend.append(issue_gather(k, issue_idx(k)))
    for k in range(NCHUNKB):
        if k + NBUFB - 1 < NCHUNKB:
            pend.append(issue_gather(k + NBUFB - 1,
                                     issue_idx(k + NBUFB - 1)))
        for h in pend.pop(0):
            h.wait()
        compute(k)
        ob = slots[k % NBUFB][3]
        base = base0 + k * CHB
        for c in range(NCTX):
            pltpu.sync_copy(ob.at[pl.ds(c * CHB, CHB)],
                            out_hbm.at[pl.ds(c * B + base, CHB)])


def _scratch_b():
    per_slot = [
        pltpu.VMEM((CHB * NCTX,), jnp.int32),
        pltpu.VMEM((CHB * NCTX, PEMB), jnp.float32),
        pltpu.VMEM((CHB * EMB,), jnp.float32),
        pltpu.VMEM((CHB * NCTX,), jnp.float32),
        pltpu.SemaphoreType.DMA,
    ]
    return per_slot * NBUFB


_SC_PARAMS = pltpu.CompilerParams(
    needs_layout_passes=False, use_tc_tiling_on_sc=True)


@jax.jit
def kernel(central_items, central_side_informations, context_items,
           item_embedding_in, item_embedding_out, weights_table, side_tables):
    ci = central_items.astype(jnp.int32)
    csi = central_side_informations.astype(jnp.int32).reshape(-1)
    ctxf = context_items.astype(jnp.int32).T.reshape(-1)
    w0 = weights_table[:, 0]
    w1 = weights_table[:, 1]
    w2 = weights_table[:, 2]

    side2 = _widen_side(jnp.transpose(side_tables, (0, 2, 1)), SIDE_VOCAB)
    ein2 = _widen_table(item_embedding_in.T, NUM_ITEMS)

    mesh = plsc.VectorSubcoreMesh(
        core_axis_name="c", subcore_axis_name="s",
        num_cores=NC, num_subcores=NS)
    run_a = pl.kernel(
        _body_a,
        out_type=jax.ShapeDtypeStruct((B * EMB,), jnp.float32),
        mesh=mesh, scratch_types=_scratch_a(), compiler_params=_SC_PARAMS)
    hid = run_a(ci, csi, ein2, w0, w1, w2, side2)

    eout2 = _widen_table(item_embedding_out.T, NUM_ITEMS)

    run_b = pl.kernel(
        _body_b,
        out_type=jax.ShapeDtypeStruct((NCTX * B,), jnp.float32),
        mesh=mesh, scratch_types=_scratch_b(), compiler_params=_SC_PARAMS)
    out = run_b(ctxf, eout2, hid)
    return out.reshape(NCTX, B).T


# speedup vs baseline: 1.0105x; 1.0105x over previous
import jax
import jax.numpy as jnp
from jax import lax
from jax.experimental import pallas as pl
from jax.experimental.pallas import tpu as pltpu
from jax.experimental.pallas import tpu_sc as plsc

NUM_ITEMS = 1000000
SIDE_VOCAB = 100000
N_SIDE = 2
EMB = 64
B = 16384
NCTX = 5

NC = 2
NS = 16
L = 16
NW = NC * NS
BW = B // NW
CH = 32
NCHUNK = BW // CH
NBUF = 3
PEMB = 2 * EMB


def _widen_body(x_ref, o_ref):
    o_ref[:, 0:EMB] = jnp.swapaxes(x_ref[...], 0, 1)


def _widen_table(xT, v, bc=16384):
    return pl.pallas_call(
        _widen_body,
        grid=(pl.cdiv(v, bc),),
        in_specs=[pl.BlockSpec((EMB, bc), lambda c: (0, c))],
        out_specs=pl.BlockSpec((bc, PEMB), lambda c: (c, 0)),
        out_shape=jax.ShapeDtypeStruct((v, PEMB), jnp.float32),
    )(xT)


def _widen_side_body(x_ref, o_ref):
    o_ref[0, :, 0:EMB] = jnp.swapaxes(x_ref[0], 0, 1)


def _widen_side(sT, v, bc=16384):
    return pl.pallas_call(
        _widen_side_body,
        grid=(N_SIDE, pl.cdiv(v, bc)),
        in_specs=[pl.BlockSpec((1, EMB, bc), lambda j, c: (j, 0, c))],
        out_specs=pl.BlockSpec((1, bc, PEMB), lambda j, c: (j, c, 0)),
        out_shape=jax.ShapeDtypeStruct((N_SIDE, v, PEMB), jnp.float32),
    )(sT)


def _softmax3(w0, w1, w2):
    m = jnp.maximum(w0, jnp.maximum(w1, w2))
    e0 = jnp.exp(w0 - m)
    e1 = jnp.exp(w1 - m)
    e2 = jnp.exp(w2 - m)
    s = e0 + e1 + e2
    return e0 / s, e1 / s, e2 / s


def _body_a(ci_hbm, csi_hbm, ein_hbm, w0_hbm, w1_hbm, w2_hbm, side_hbm,
            hid_hbm, *scratch):
    per = 9
    slots = [scratch[i * per:(i + 1) * per] for i in range(NBUF)]
    wid = lax.axis_index("s") * NC + lax.axis_index("c")
    base0 = wid * BW
    iota16 = lax.iota(jnp.int32, L)

    def issue_idx(k):
        ii, is0, is1, ri, rs0, rs1, w3, hb, sem = slots[k % NBUF]
        base = base0 + k * CH
        hs = [
            pltpu.make_async_copy(ci_hbm.at[pl.ds(base, CH)], ii, sem),
            pltpu.make_async_copy(csi_hbm.at[pl.ds(base, CH)], is0, sem),
            pltpu.make_async_copy(csi_hbm.at[pl.ds(B + base, CH)], is1,
                                  sem),
        ]
        for h in hs:
            h.start()
        return hs

    def issue_gather(k, idx_pending):
        ii, is0, is1, ri, rs0, rs1, w3, hb, sem = slots[k % NBUF]
        for h in idx_pending:
            h.wait()
        hs = [
            pltpu.make_async_copy(ein_hbm.at[ii], ri, sem),
            pltpu.make_async_copy(side_hbm.at[0].at[is0], rs0, sem),
            pltpu.make_async_copy(side_hbm.at[1].at[is1], rs1, sem),
            pltpu.make_async_copy(w0_hbm.at[ii], w3.at[pl.ds(0, CH)], sem),
            pltpu.make_async_copy(w1_hbm.at[ii], w3.at[pl.ds(CH, CH)], sem),
            pltpu.make_async_copy(w2_hbm.at[ii], w3.at[pl.ds(2 * CH, CH)],
                                  sem),
        ]
        for h in hs:
            h.start()
        return hs

    def compute(k):
        ii, is0, is1, ri, rs0, rs1, w3, hb, sem = slots[k % NBUF]

        def group(g, carry):
            o = g * L
            lane = jnp.full((L,), o, jnp.int32) + iota16
            lane64 = lane * EMB
            p0, p1, p2 = _softmax3(
                w3[pl.ds(o, L)], w3[pl.ds(CH + o, L)],
                w3[pl.ds(2 * CH + o, L)])

            def dbody(d, carry2):
                dv = jnp.full((L,), d, jnp.int32)
                h = (p0 * plsc.load_gather(ri, [lane, dv])
                     + p1 * plsc.load_gather(rs0, [lane, dv])
                     + p2 * plsc.load_gather(rs1, [lane, dv]))
                plsc.store_scatter(hb, [lane64 + dv], h)
                return carry2

            lax.fori_loop(0, EMB, dbody, 0)
            return carry

        lax.fori_loop(0, CH // L, group, 0)

    pend = [issue_gather(0, issue_idx(0))]
    for k in range(1, NBUF - 1):
        pend.append(issue_gather(k, issue_idx(k)))
    for k in range(NCHUNK):
        if k + NBUF - 1 < NCHUNK:
            pend.append(issue_gather(k + NBUF - 1,
                                     issue_idx(k + NBUF - 1)))
        for h in pend.pop(0):
            h.wait()
        compute(k)
        hb = slots[k % NBUF][7]
        pltpu.sync_copy(
            hb, hid_hbm.at[pl.ds((base0 + k * CH) * EMB, CH * EMB)])


def _scratch_a():
    per_slot = [
        pltpu.VMEM((CH,), jnp.int32),
        pltpu.VMEM((CH,), jnp.int32),
        pltpu.VMEM((CH,), jnp.int32),
        pltpu.VMEM((CH, PEMB), jnp.float32),
        pltpu.VMEM((CH, PEMB), jnp.float32),
        pltpu.VMEM((CH, PEMB), jnp.float32),
        pltpu.VMEM((CH * 3,), jnp.float32),
        pltpu.VMEM((CH * EMB,), jnp.float32),
        pltpu.SemaphoreType.DMA,
    ]
    return per_slot * NBUF


CHB = 64
NBUFB = 2
NCHUNKB = BW // CHB


def _body_b(ctx_hbm, eout_hbm, hid_hbm, out_hbm, *scratch):
    per = 5
    slots = [scratch[i * per:(i + 1) * per] for i in range(NBUFB)]
    wid = lax.axis_index("s") * NC + lax.axis_index("c")
    base0 = wid * BW
    iota16 = lax.iota(jnp.int32, L)

    def issue_idx(k):
        icf, rc, hbv, ob, sem = slots[k % NBUFB]
        base = base0 + k * CHB
        hs = [pltpu.make_async_copy(
            hid_hbm.at[pl.ds(base * EMB, CHB * EMB)], hbv, sem)]
        for c in range(NCTX):
            hs.append(pltpu.make_async_copy(
                ctx_hbm.at[pl.ds(c * B + base, CHB)],
                icf.at[pl.ds(c * CHB, CHB)], sem))
        for h in hs:
            h.start()
        return hs

    def issue_gather(k, idx_pending):
        icf, rc, hbv, ob, sem = slots[k % NBUFB]
        for h in idx_pending:
            h.wait()
        hs = []
        for c in range(NCTX):
            hs.append(pltpu.make_async_copy(
                eout_hbm.at[icf.at[pl.ds(c * CHB, CHB)]],
                rc.at[pl.ds(c * CHB, CHB)], sem))
        for h in hs:
            h.start()
        return hs

    def compute(k):
        icf, rc, hbv, ob, sem = slots[k % NBUFB]
        zf = jnp.zeros((L,), jnp.float32)

        def group(g, carry):
            o = g * L
            lane = jnp.full((L,), o, jnp.int32) + iota16
            lane64 = lane * EMB

            def dbody(d, accs):
                dv = jnp.full((L,), d, jnp.int32)
                hv = plsc.load_gather(hbv, [lane64 + dv])
                return tuple(
                    accs[c] + hv * plsc.load_gather(
                        rc, [jnp.full((L,), c * CHB, jnp.int32) + lane, dv])
                    for c in range(NCTX))

            accs = lax.fori_loop(0, EMB, dbody, (zf,) * NCTX)
            for c in range(NCTX):
                sig = 1.0 / (1.0 + jnp.exp(-accs[c]))
                plsc.store_scatter(
                    ob, [jnp.full((L,), c * CHB, jnp.int32) + lane], sig)
            return carry

        lax.fori_loop(0, CHB // L, group, 0)

    pend = [issue_gather(0, issue_idx(0))]
    for k in range(1, NBUFB - 1):
        pend.append(issue_gather(k, issue_idx(k)))
    for k in range(NCHUNKB):
        if k + NBUFB - 1 < NCHUNKB:
            pend.append(issue_gather(k + NBUFB - 1,
                                     issue_idx(k + NBUFB - 1)))
        for h in pend.pop(0):
            h.wait()
        compute(k)
        ob = slots[k % NBUFB][3]
        base = base0 + k * CHB
        for c in range(NCTX):
            pltpu.sync_copy(ob.at[pl.ds(c * CHB, CHB)],
                            out_hbm.at[pl.ds(c * B + base, CHB)])


def _scratch_b():
    per_slot = [
        pltpu.VMEM((CHB * NCTX,), jnp.int32),
        pltpu.VMEM((CHB * NCTX, PEMB), jnp.float32),
        pltpu.VMEM((CHB * EMB,), jnp.float32),
        pltpu.VMEM((CHB * NCTX,), jnp.float32),
        pltpu.SemaphoreType.DMA,
    ]
    return per_slot * NBUFB


_SC_PARAMS = pltpu.CompilerParams(
    needs_layout_passes=False, use_tc_tiling_on_sc=True)


@jax.jit
def kernel(central_items, central_side_informations, context_items,
           item_embedding_in, item_embedding_out, weights_table, side_tables):
    ci = central_items.astype(jnp.int32)
    csi = central_side_informations.astype(jnp.int32).reshape(-1)
    ctxf = context_items.astype(jnp.int32).T.reshape(-1)
    w0 = weights_table[:, 0]
    w1 = weights_table[:, 1]
    w2 = weights_table[:, 2]

    side2 = _widen_side(jnp.transpose(side_tables, (0, 2, 1)), SIDE_VOCAB)
    ein2 = _widen_table(item_embedding_in.T, NUM_ITEMS)

    mesh = plsc.VectorSubcoreMesh(
        core_axis_name="c", subcore_axis_name="s",
        num_cores=NC, num_subcores=NS)
    run_a = pl.kernel(
        _body_a,
        out_type=jax.ShapeDtypeStruct((B * EMB,), jnp.float32),
        mesh=mesh, scratch_types=_scratch_a(), compiler_params=_SC_PARAMS)
    hid = run_a(ci, csi, ein2, w0, w1, w2, side2)

    eout2 = _widen_table(item_embedding_out.T, NUM_ITEMS)

    run_b = pl.kernel(
        _body_b,
        out_type=jax.ShapeDtypeStruct((NCTX * B,), jnp.float32),
        mesh=mesh, scratch_types=_scratch_b(), compiler_params=_SC_PARAMS)
    out = run_b(ctxf, eout2, hid)
    return out.reshape(NCTX, B).T
